# baseline (speedup 1.0000x reference)
import jax
import jax.numpy as jnp
from jax.experimental import pallas as pl
from jax.experimental.pallas import tpu as pltpu

_B, _N, _DIM = 2, 2048, 64
_H, _DH = 8, 64
_CBS = 64
_SBS = 64
_NSEL = 8
_WIN = 128
_NB = _N // _CBS
_QT = 64
_BH = _B * _H


def _attn_body(q_ref, k_ref, v_ref, ckf_ref, cvf_ref, g_ref, o_ref):
    i = pl.program_id(1)
    q = q_ref[0]
    scale = _DH ** -0.5
    rows = i * _QT + jax.lax.broadcasted_iota(jnp.int32, (_QT, 1), 0)

    ckf = ckf_ref[0]
    csim = jnp.dot(q, ckf.T, preferred_element_type=jnp.float32) * scale
    col = jax.lax.broadcasted_iota(jnp.int32, (_QT, _NB + 1), 1)
    blk_end = col * _CBS - 1
    csim = jnp.where(blk_end <= rows, csim, -1e9)
    cmax = jnp.max(csim, axis=-1, keepdims=True)
    cexp = jnp.exp(csim - cmax)
    cattn = cexp / jnp.sum(cexp, axis=-1, keepdims=True)
    c_out = jnp.dot(cattn, cvf_ref[0], preferred_element_type=jnp.float32)

    imp = cattn[:, 1:]
    blkids = jax.lax.broadcasted_iota(jnp.int32, (_QT, _NB), 1)
    imp = jnp.where(blkids == rows // _SBS, 1e4, imp)
    a = imp[:, :, None]
    b = imp[:, None, :]
    jj = jax.lax.broadcasted_iota(jnp.int32, (_QT, _NB, _NB), 1)
    kk = jax.lax.broadcasted_iota(jnp.int32, (_QT, _NB, _NB), 2)
    beats = (b > a) | ((b == a) & (kk < jj))
    rank = jnp.sum(beats.astype(jnp.float32), axis=2)
    self_f = (rank < _NSEL).astype(jnp.float32)

    keyblk = jax.lax.broadcasted_iota(jnp.int32, (_NB, _N), 1) // _CBS
    expand = (keyblk == jax.lax.broadcasted_iota(jnp.int32, (_NB, _N), 0))
    sel_cols = jnp.dot(self_f, expand.astype(jnp.float32),
                       preferred_element_type=jnp.float32)

    k = k_ref[0]
    v = v_ref[0]
    s = jnp.dot(q, k.T, preferred_element_type=jnp.float32) * scale
    kpos = jax.lax.broadcasted_iota(jnp.int32, (_QT, _N), 1)
    causal = kpos <= rows

    fmask = causal & (sel_cols > 0.5)
    fs = jnp.where(fmask, s, -1e9)
    fmax = jnp.max(fs, axis=-1, keepdims=True)
    fexp = jnp.exp(fs - fmax)
    f_out = jnp.dot(fexp / jnp.sum(fexp, axis=-1, keepdims=True), v,
                    preferred_element_type=jnp.float32)

    wmask = causal & (kpos > rows - _WIN)
    ws = jnp.where(wmask, s, -1e9)
    wmax = jnp.max(ws, axis=-1, keepdims=True)
    wexp = jnp.exp(ws - wmax)
    s_out = jnp.dot(wexp / jnp.sum(wexp, axis=-1, keepdims=True), v,
                    preferred_element_type=jnp.float32)

    g = g_ref[0]
    o_ref[0] = (g[:, 0:1] * c_out + g[:, 1:2] * f_out + g[:, 2:3] * s_out)


def _fused_attention(q, k, v, ckf, cvf, gates):
    return pl.pallas_call(
        _attn_body,
        grid=(_BH, _N // _QT),
        in_specs=[
            pl.BlockSpec((1, _QT, _DH), lambda bh, i: (bh, i, 0)),
            pl.BlockSpec((1, _N, _DH), lambda bh, i: (bh, 0, 0)),
            pl.BlockSpec((1, _N, _DH), lambda bh, i: (bh, 0, 0)),
            pl.BlockSpec((1, _NB + 1, _DH), lambda bh, i: (bh, 0, 0)),
            pl.BlockSpec((1, _NB + 1, _DH), lambda bh, i: (bh, 0, 0)),
            pl.BlockSpec((1, _QT, 3), lambda bh, i: (bh, i, 0)),
        ],
        out_specs=pl.BlockSpec((1, _QT, _DH), lambda bh, i: (bh, i, 0)),
        out_shape=jax.ShapeDtypeStruct((_BH, _N, _DH), jnp.float32),
        compiler_params=pltpu.CompilerParams(
            dimension_semantics=("parallel", "arbitrary"),
        ),
    )(q, k, v, ckf, cvf, gates)


def kernel(x, W_emb, b_emb, g_norm, W_q, W_k, W_v, k_pos, v_pos, Wc_k, Wc_v,
           mem_ck, mem_cv, W_gate, b_gate, W_o, W1, b1, W2, b2):
    B, N, DIM, H, DH, NB, CBS = _B, _N, _DIM, _H, _DH, _NB, _CBS

    xe = x[..., None] * W_emb[0] + b_emb
    h = xe * jax.lax.rsqrt(jnp.mean(xe * xe, axis=-1, keepdims=True) + 1e-6)
    h = h * g_norm

    def proj(W):
        return (h @ W).reshape(B, N, H, DH).transpose(0, 2, 1, 3)

    q, k, v = proj(W_q), proj(W_k), proj(W_v)

    kb = k.reshape(B, H, NB, CBS, DH) + k_pos
    vb = v.reshape(B, H, NB, CBS, DH) + v_pos
    ck = kb.reshape(B, H, NB, CBS * DH) @ Wc_k
    cv = vb.reshape(B, H, NB, CBS * DH) @ Wc_v
    ckf = jnp.concatenate([jnp.broadcast_to(mem_ck, (B, H, 1, DH)), ck], axis=2)
    cvf = jnp.concatenate([jnp.broadcast_to(mem_cv, (B, H, 1, DH)), cv], axis=2)

    gates = jax.nn.sigmoid(xe @ W_gate + b_gate)
    gates = gates.reshape(B, N, 3, H).transpose(0, 3, 1, 2)

    out = _fused_attention(
        q.reshape(_BH, N, DH), k.reshape(_BH, N, DH), v.reshape(_BH, N, DH),
        ckf.reshape(_BH, NB + 1, DH), cvf.reshape(_BH, NB + 1, DH),
        gates.reshape(_BH, N, 3),
    )

    out = out.reshape(B, H, N, DH).transpose(0, 2, 1, 3).reshape(B, N, H * DH)
    out = out @ W_o
    pooled = out.mean(axis=1)
    h1 = jax.nn.gelu(pooled @ W1 + b1, approximate=False)
    return h1 @ W2 + b2


# speedup vs baseline: 1.1486x; 1.1486x over previous
import jax
import jax.numpy as jnp
from jax.experimental import pallas as pl
from jax.experimental.pallas import tpu as pltpu

_B, _N, _DIM = 2, 2048, 64
_H, _DH = 8, 64
_CBS = 64
_SBS = 64
_NSEL = 8
_WIN = 128
_NB = _N // _CBS
_QT = 128
_KT = 512
_WT = 2 * _QT
_BH = _B * _H


def _attn_body(q_ref, k_ref, v_ref, ckf_ref, cvf_ref, g_ref, o_ref):
    i = pl.program_id(1)
    q = q_ref[0]
    scale = _DH ** -0.5
    rows = i * _QT + jax.lax.broadcasted_iota(jnp.int32, (_QT, 1), 0)

    ckf = ckf_ref[0]
    csim = jnp.dot(q, ckf.T, preferred_element_type=jnp.float32) * scale
    col = jax.lax.broadcasted_iota(jnp.int32, (_QT, _NB + 1), 1)
    blk_end = col * _CBS - 1
    csim = jnp.where(blk_end <= rows, csim, -1e9)
    cmax = jnp.max(csim, axis=-1, keepdims=True)
    cexp = jnp.exp(csim - cmax)
    cattn = cexp / jnp.sum(cexp, axis=-1, keepdims=True)
    c_out = jnp.dot(cattn, cvf_ref[0], preferred_element_type=jnp.float32)

    imp = cattn[:, 1:]
    blkids = jax.lax.broadcasted_iota(jnp.int32, (_QT, _NB), 1)
    imp = jnp.where(blkids == rows // _SBS, 1e4, imp)
    a = imp[:, :, None]
    b = imp[:, None, :]
    jj = jax.lax.broadcasted_iota(jnp.int32, (_QT, _NB, _NB), 1)
    kk = jax.lax.broadcasted_iota(jnp.int32, (_QT, _NB, _NB), 2)
    beats = (b > a) | ((b == a) & (kk < jj))
    rank = jnp.sum(beats.astype(jnp.float32), axis=2)
    sel_f = (rank < _NSEL).astype(jnp.float32)

    nchunks = (_QT * (i + 1) + _KT - 1) // _KT

    def chunk(c, carry):
        m, l, acc = carry
        k_c = k_ref[0, pl.ds(c * _KT, _KT), :]
        v_c = v_ref[0, pl.ds(c * _KT, _KT), :]
        s = jnp.dot(q, k_c.T, preferred_element_type=jnp.float32) * scale
        kpos = c * _KT + jax.lax.broadcasted_iota(jnp.int32, (_QT, _KT), 1)
        kblk = kpos // _CBS
        nb_iota = jax.lax.broadcasted_iota(jnp.int32, (_NB, _KT), 0)
        expand = (nb_iota == (c * _KT // _CBS)
                  + jax.lax.broadcasted_iota(jnp.int32, (_NB, _KT), 1) // _CBS)
        sel_cols = jnp.dot(sel_f, expand.astype(jnp.float32),
                           preferred_element_type=jnp.float32)
        mask = (kpos <= rows) & (sel_cols > 0.5)
        s = jnp.where(mask, s, -1e9)
        m_new = jnp.maximum(m, jnp.max(s, axis=-1, keepdims=True))
        p = jnp.exp(s - m_new)
        alpha = jnp.exp(m - m_new)
        l = l * alpha + jnp.sum(p, axis=-1, keepdims=True)
        acc = acc * alpha + jnp.dot(p, v_c, preferred_element_type=jnp.float32)
        return m_new, l, acc

    m0 = jnp.full((_QT, 1), -1e30, jnp.float32)
    l0 = jnp.zeros((_QT, 1), jnp.float32)
    a0 = jnp.zeros((_QT, _DH), jnp.float32)
    m, l, acc = jax.lax.fori_loop(0, nchunks, chunk, (m0, l0, a0))
    f_out = acc / l

    wstart = jnp.maximum(i - 1, 0) * _QT
    k_w = k_ref[0, pl.ds(wstart, _WT), :]
    v_w = v_ref[0, pl.ds(wstart, _WT), :]
    ws = jnp.dot(q, k_w.T, preferred_element_type=jnp.float32) * scale
    wpos = wstart + jax.lax.broadcasted_iota(jnp.int32, (_QT, _WT), 1)
    wmask = (wpos <= rows) & (wpos > rows - _WIN)
    ws = jnp.where(wmask, ws, -1e9)
    wmax = jnp.max(ws, axis=-1, keepdims=True)
    wexp = jnp.exp(ws - wmax)
    s_out = jnp.dot(wexp / jnp.sum(wexp, axis=-1, keepdims=True), v_w,
                    preferred_element_type=jnp.float32)

    g = g_ref[0]
    o_ref[0] = (g[:, 0:1] * c_out + g[:, 1:2] * f_out + g[:, 2:3] * s_out)


def _fused_attention(q, k, v, ckf, cvf, gates):
    return pl.pallas_call(
        _attn_body,
        grid=(_BH, _N // _QT),
        in_specs=[
            pl.BlockSpec((1, _QT, _DH), lambda bh, i: (bh, i, 0)),
            pl.BlockSpec((1, _N, _DH), lambda bh, i: (bh, 0, 0)),
            pl.BlockSpec((1, _N, _DH), lambda bh, i: (bh, 0, 0)),
            pl.BlockSpec((1, _NB + 1, _DH), lambda bh, i: (bh, 0, 0)),
            pl.BlockSpec((1, _NB + 1, _DH), lambda bh, i: (bh, 0, 0)),
            pl.BlockSpec((1, _QT, 3), lambda bh, i: (bh, i, 0)),
        ],
        out_specs=pl.BlockSpec((1, _QT, _DH), lambda bh, i: (bh, i, 0)),
        out_shape=jax.ShapeDtypeStruct((_BH, _N, _DH), jnp.float32),
        compiler_params=pltpu.CompilerParams(
            dimension_semantics=("parallel", "arbitrary"),
        ),
    )(q, k, v, ckf, cvf, gates)


def kernel(x, W_emb, b_emb, g_norm, W_q, W_k, W_v, k_pos, v_pos, Wc_k, Wc_v,
           mem_ck, mem_cv, W_gate, b_gate, W_o, W1, b1, W2, b2):
    B, N, DIM, H, DH, NB, CBS = _B, _N, _DIM, _H, _DH, _NB, _CBS

    xe = x[..., None] * W_emb[0] + b_emb
    h = xe * jax.lax.rsqrt(jnp.mean(xe * xe, axis=-1, keepdims=True) + 1e-6)
    h = h * g_norm

    def proj(W):
        return (h @ W).reshape(B, N, H, DH).transpose(0, 2, 1, 3)

    q, k, v = proj(W_q), proj(W_k), proj(W_v)

    kb = k.reshape(B, H, NB, CBS, DH) + k_pos
    vb = v.reshape(B, H, NB, CBS, DH) + v_pos
    ck = kb.reshape(B, H, NB, CBS * DH) @ Wc_k
    cv = vb.reshape(B, H, NB, CBS * DH) @ Wc_v
    ckf = jnp.concatenate([jnp.broadcast_to(mem_ck, (B, H, 1, DH)), ck], axis=2)
    cvf = jnp.concatenate([jnp.broadcast_to(mem_cv, (B, H, 1, DH)), cv], axis=2)

    gates = jax.nn.sigmoid(xe @ W_gate + b_gate)
    gates = gates.reshape(B, N, 3, H).transpose(0, 3, 1, 2)

    out = _fused_attention(
        q.reshape(_BH, N, DH), k.reshape(_BH, N, DH), v.reshape(_BH, N, DH),
        ckf.reshape(_BH, NB + 1, DH), cvf.reshape(_BH, NB + 1, DH),
        gates.reshape(_BH, N, 3),
    )

    out = out.reshape(B, H, N, DH).transpose(0, 2, 1, 3).reshape(B, N, H * DH)
    out = out @ W_o
    pooled = out.mean(axis=1)
    h1 = jax.nn.gelu(pooled @ W1 + b1, approximate=False)
    return h1 @ W2 + b2


# speedup vs baseline: 2.5243x; 2.1978x over previous
import jax
import jax.numpy as jnp
from jax.experimental import pallas as pl
from jax.experimental.pallas import tpu as pltpu

_B, _N, _DIM = 2, 2048, 64
_H, _DH = 8, 64
_CBS = 64
_SBS = 64
_NSEL = 8
_WIN = 128
_NB = _N // _CBS
_QT = 256
_KT = 512
_WT = _WIN + _QT
_BH = _B * _H

_NT = (((1,), (1,)), ((), ()))
_TN = (((0,), (0,)), ((), ()))


def _attn_body(q_ref, k_ref, v_ref, ckf_ref, cvf_ref, g_ref, o_ref):
    i = pl.program_id(1)
    q = q_ref[0]
    scale = _DH ** -0.5
    rows = i * _QT + jax.lax.broadcasted_iota(jnp.int32, (_QT, 1), 0)

    ckf = ckf_ref[0]
    csim = jax.lax.dot_general(q, ckf, _NT,
                               preferred_element_type=jnp.float32) * scale
    col = jax.lax.broadcasted_iota(jnp.int32, (_QT, _NB + 1), 1)
    csim = jnp.where(col * _CBS - 1 <= rows, csim, -1e9)
    cmax = jnp.max(csim, axis=-1, keepdims=True)
    cexp = jnp.exp(csim - cmax)
    cattn = cexp / jnp.sum(cexp, axis=-1, keepdims=True)
    c_out = jnp.dot(cattn, cvf_ref[0], preferred_element_type=jnp.float32)

    ck1 = ckf[1:, :]
    rsim = jax.lax.dot_general(ck1, q, _NT,
                               preferred_element_type=jnp.float32) * scale
    jrow = jax.lax.broadcasted_iota(jnp.int32, (_NB, _QT), 0)
    ncol = i * _QT + jax.lax.broadcasted_iota(jnp.int32, (_NB, _QT), 1)
    rsim = jnp.where((jrow + 1) * _CBS - 1 <= ncol, rsim, -1e9)
    rsim = jnp.where(jrow == ncol // _SBS, 1e4, rsim)
    a = rsim[:, None, :]
    b = rsim[None, :, :]
    jj = jax.lax.broadcasted_iota(jnp.int32, (_NB, _NB, 1), 0)
    kk = jax.lax.broadcasted_iota(jnp.int32, (_NB, _NB, 1), 1)
    beats = (b > a) | ((b == a) & (kk < jj))
    rank = jnp.sum(beats.astype(jnp.float32), axis=1)
    selT = (rank < _NSEL - 0.5).astype(jnp.float32)

    nchunks = (_QT * (i + 1) + _KT - 1) // _KT

    def chunk(c, carry):
        m, l, acc = carry
        k_c = k_ref[0, pl.ds(c * _KT, _KT), :]
        v_c = v_ref[0, pl.ds(c * _KT, _KT), :]
        s = jax.lax.dot_general(q, k_c, _NT,
                                preferred_element_type=jnp.float32) * scale
        colblk = (jax.lax.broadcasted_iota(jnp.int32, (_NB, _KT), 1) // _CBS
                  + c * (_KT // _CBS))
        ex = (jax.lax.broadcasted_iota(jnp.int32, (_NB, _KT), 0)
              == colblk).astype(jnp.float32)
        sel_cols = jax.lax.dot_general(selT, ex, _TN,
                                       preferred_element_type=jnp.float32)
        kpos = c * _KT + jax.lax.broadcasted_iota(jnp.int32, (_QT, _KT), 1)
        mask = (kpos <= rows) & (sel_cols > 0.5)
        s = jnp.where(mask, s, -1e9)
        m_new = jnp.maximum(m, jnp.max(s, axis=-1, keepdims=True))
        p = jnp.exp(s - m_new)
        alpha = jnp.exp(m - m_new)
        l = l * alpha + jnp.sum(p, axis=-1, keepdims=True)
        acc = acc * alpha + jnp.dot(p, v_c, preferred_element_type=jnp.float32)
        return m_new, l, acc

    m0 = jnp.full((_QT, 1), -1e30, jnp.float32)
    l0 = jnp.zeros((_QT, 1), jnp.float32)
    a0 = jnp.zeros((_QT, _DH), jnp.float32)
    m, l, acc = jax.lax.fori_loop(0, nchunks, chunk, (m0, l0, a0))
    f_out = acc / l

    wstart = jnp.maximum(i * _QT - _WIN, 0)
    k_w = k_ref[0, pl.ds(wstart, _WT), :]
    v_w = v_ref[0, pl.ds(wstart, _WT), :]
    ws = jax.lax.dot_general(q, k_w, _NT,
                             preferred_element_type=jnp.float32) * scale
    wpos = wstart + jax.lax.broadcasted_iota(jnp.int32, (_QT, _WT), 1)
    wmask = (wpos <= rows) & (wpos > rows - _WIN)
    ws = jnp.where(wmask, ws, -1e9)
    wmax = jnp.max(ws, axis=-1, keepdims=True)
    wexp = jnp.exp(ws - wmax)
    s_out = jnp.dot(wexp / jnp.sum(wexp, axis=-1, keepdims=True), v_w,
                    preferred_element_type=jnp.float32)

    g = g_ref[0]
    o_ref[0] = (g[:, 0:1] * c_out + g[:, 1:2] * f_out + g[:, 2:3] * s_out)


def _fused_attention(q, k, v, ckf, cvf, gates):
    return pl.pallas_call(
        _attn_body,
        grid=(_BH, _N // _QT),
        in_specs=[
            pl.BlockSpec((1, _QT, _DH), lambda bh, i: (bh, i, 0)),
            pl.BlockSpec((1, _N, _DH), lambda bh, i: (bh, 0, 0)),
            pl.BlockSpec((1, _N, _DH), lambda bh, i: (bh, 0, 0)),
            pl.BlockSpec((1, _NB + 1, _DH), lambda bh, i: (bh, 0, 0)),
            pl.BlockSpec((1, _NB + 1, _DH), lambda bh, i: (bh, 0, 0)),
            pl.BlockSpec((1, _QT, 3), lambda bh, i: (bh, i, 0)),
        ],
        out_specs=pl.BlockSpec((1, _QT, _DH), lambda bh, i: (bh, i, 0)),
        out_shape=jax.ShapeDtypeStruct((_BH, _N, _DH), jnp.float32),
        compiler_params=pltpu.CompilerParams(
            dimension_semantics=("parallel", "arbitrary"),
        ),
    )(q, k, v, ckf, cvf, gates)


def kernel(x, W_emb, b_emb, g_norm, W_q, W_k, W_v, k_pos, v_pos, Wc_k, Wc_v,
           mem_ck, mem_cv, W_gate, b_gate, W_o, W1, b1, W2, b2):
    B, N, DIM, H, DH, NB, CBS = _B, _N, _DIM, _H, _DH, _NB, _CBS

    xe = x[..., None] * W_emb[0] + b_emb
    h = xe * jax.lax.rsqrt(jnp.mean(xe * xe, axis=-1, keepdims=True) + 1e-6)
    h = h * g_norm

    def proj(W):
        return (h @ W).reshape(B, N, H, DH).transpose(0, 2, 1, 3)

    q, k, v = proj(W_q), proj(W_k), proj(W_v)

    kb = k.reshape(B, H, NB, CBS, DH) + k_pos
    vb = v.reshape(B, H, NB, CBS, DH) + v_pos
    ck = kb.reshape(B, H, NB, CBS * DH) @ Wc_k
    cv = vb.reshape(B, H, NB, CBS * DH) @ Wc_v
    ckf = jnp.concatenate([jnp.broadcast_to(mem_ck, (B, H, 1, DH)), ck], axis=2)
    cvf = jnp.concatenate([jnp.broadcast_to(mem_cv, (B, H, 1, DH)), cv], axis=2)

    gates = jax.nn.sigmoid(xe @ W_gate + b_gate)
    gates = gates.reshape(B, N, 3, H).transpose(0, 3, 1, 2)

    out = _fused_attention(
        q.reshape(_BH, N, DH), k.reshape(_BH, N, DH), v.reshape(_BH, N, DH),
        ckf.reshape(_BH, NB + 1, DH), cvf.reshape(_BH, NB + 1, DH),
        gates.reshape(_BH, N, 3),
    )

    out = out.reshape(B, H, N, DH).transpose(0, 2, 1, 3).reshape(B, N, H * DH)
    out = out @ W_o
    pooled = out.mean(axis=1)
    h1 = jax.nn.gelu(pooled @ W1 + b1, approximate=False)
    return h1 @ W2 + b2
